# baseline (speedup 1.0000x reference)
import functools

import jax
import jax.numpy as jnp
from jax import lax
from jax.experimental import pallas as pl
from jax.experimental.pallas import tpu as pltpu
from jax.experimental.pallas import tpu_sc as plsc

SCALE_ = 8.0

_info = plsc.get_sparse_core_info()
_NC, _NS, _L = _info.num_cores, _info.num_subcores, _info.num_lanes
_NW = _NC * _NS


def _make_sc_embed(B, V, D, CHUNK):
    assert B % _NW == 0
    b_per_w = B // _NW
    assert b_per_w % CHUNK == 0
    n_chunks = b_per_w // CHUNK
    assert CHUNK % 16 == 0
    groups = CHUNK // _L

    mesh = plsc.VectorSubcoreMesh(core_axis_name="c", subcore_axis_name="s")

    @functools.partial(
        pl.kernel,
        mesh=mesh,
        out_type=jax.ShapeDtypeStruct((B, D), jnp.float32),
        scratch_types=[
            pltpu.VMEM((b_per_w,), jnp.int32),
            pltpu.VMEM((b_per_w,), jnp.int32),
            pltpu.VMEM((CHUNK, 2 * D), jnp.float32),
            pltpu.VMEM((CHUNK, 2 * D), jnp.float32),
            pltpu.VMEM((CHUNK, D), jnp.float32),
            pltpu.VMEM((CHUNK, D), jnp.float32),
            pltpu.SemaphoreType.DMA,
            pltpu.SemaphoreType.DMA,
            pltpu.SemaphoreType.DMA,
            pltpu.SemaphoreType.DMA,
        ],
        compiler_params=pltpu.CompilerParams(
            use_tc_tiling_on_sc=False, needs_layout_passes=False
        ),
    )
    def k(idx_hbm, table2_hbm, out_hbm, idx_all, pair_all,
          rows0, rows1, outb0, outb1, g0, g1, s0, s1):
        wid = lax.axis_index("s") * _NC + lax.axis_index("c")
        base = wid * b_per_w
        pltpu.sync_copy(idx_hbm.at[pl.ds(base, b_per_w)], idx_all)

        @plsc.parallel_loop(0, b_per_w // _L, unroll=8)
        def _pair(i):
            sl = pl.ds(i * _L, _L)
            pair_all[sl] = lax.shift_right_logical(idx_all[sl], 1)

        rows = (rows0, rows1)
        outs = (outb0, outb1)
        gsem = (g0, g1)
        ssem = (s0, s1)
        iota = lax.iota(jnp.int32, _L)

        def gather_start(cc):
            b = cc % 2
            return pltpu.async_copy(
                table2_hbm.at[pair_all.at[pl.ds(cc * CHUNK, CHUNK)]],
                rows[b],
                gsem[b],
            )

        def scatter_start(cc):
            b = cc % 2
            return pltpu.async_copy(
                outs[b],
                out_hbm.at[pl.ds(base + cc * CHUNK, CHUNK)],
                ssem[b],
            )

        gathers = {0: gather_start(0)}
        scatters = {}
        for cc in range(n_chunks):
            b = cc % 2
            gathers[cc].wait()
            if cc >= 1:
                scatters[cc - 1].wait()
            if cc + 1 < n_chunks:
                gathers[cc + 1] = gather_start(cc + 1)

            rbuf = rows[b]
            obuf = outs[b]

            @plsc.parallel_loop(0, groups)
            def _grp(g, _rbuf=rbuf, _obuf=obuf, _cc=cc):
                rv = g * _L + iota
                iv = idx_all[pl.ds(_cc * CHUNK + g * _L, _L)]
                h64 = (iv & 1) * D

                @plsc.parallel_loop(0, D, unroll=4)
                def _col(c):
                    v = plsc.load_gather(_rbuf, [rv, h64 + c])
                    plsc.store_scatter(_obuf, [rv, jnp.broadcast_to(c, (_L,))],
                                       v * SCALE_)

            scatters[cc] = scatter_start(cc)
        scatters[n_chunks - 1].wait()

    return k


@jax.jit
def kernel(x, table):
    B = x.shape[0] * x.shape[1]
    V, D = table.shape
    flat_idx = x.reshape(-1).astype(jnp.int32)
    table2 = table.reshape(V // 2, 2 * D)
    out = _make_sc_embed(B, V, D, 256)(flat_idx, table2)
    return out.reshape(x.shape[0], x.shape[1], D)


# speedup vs baseline: 1.3494x; 1.3494x over previous
import functools

import jax
import jax.numpy as jnp
from jax import lax
from jax.experimental import pallas as pl
from jax.experimental.pallas import tpu as pltpu
from jax.experimental.pallas import tpu_sc as plsc

SCALE_ = 8.0

_info = plsc.get_sparse_core_info()
_NC, _NS, _L = _info.num_cores, _info.num_subcores, _info.num_lanes
_NW = _NC * _NS


def _make_sc_embed(B, V, D, CHUNK):
    assert B % _NW == 0
    b_per_w = B // _NW
    assert b_per_w % CHUNK == 0
    n_chunks = b_per_w // CHUNK
    assert CHUNK % 16 == 0
    vregs_per_row = D // _L

    mesh = plsc.VectorSubcoreMesh(core_axis_name="c", subcore_axis_name="s")

    @functools.partial(
        pl.kernel,
        mesh=mesh,
        out_type=jax.ShapeDtypeStruct((B // 2, 2 * D), jnp.float32),
        scratch_types=[
            pltpu.VMEM((b_per_w,), jnp.int32),
            pltpu.VMEM((CHUNK, D), jnp.float32),
            pltpu.VMEM((CHUNK, D), jnp.float32),
            pltpu.VMEM((CHUNK // 2, 2 * D), jnp.float32),
            pltpu.VMEM((CHUNK // 2, 2 * D), jnp.float32),
            pltpu.SemaphoreType.DMA,
            pltpu.SemaphoreType.DMA,
            pltpu.SemaphoreType.DMA,
            pltpu.SemaphoreType.DMA,
        ],
        compiler_params=pltpu.CompilerParams(use_tc_tiling_on_sc=False),
    )
    def k(idx_hbm, table_hbm, out2_hbm, idx_all, rows0, rows1, ob0, ob1,
          g0, g1, s0, s1):
        wid = lax.axis_index("s") * _NC + lax.axis_index("c")
        base = wid * b_per_w
        pltpu.sync_copy(idx_hbm.at[pl.ds(base, b_per_w)], idx_all)

        bufs = (rows0, rows1)
        obufs = (ob0, ob1)
        gsem = (g0, g1)
        ssem = (s0, s1)

        def gather_start(cc):
            b = cc % 2
            return pltpu.async_copy(
                table_hbm.at[idx_all.at[pl.ds(cc * CHUNK, CHUNK)]],
                bufs[b],
                gsem[b],
            )

        def scatter_start(cc):
            b = cc % 2
            return pltpu.async_copy(
                obufs[b],
                out2_hbm.at[pl.ds((base + cc * CHUNK) // 2, CHUNK // 2)],
                ssem[b],
            )

        gathers = {0: gather_start(0)}
        scatters = {}
        for cc in range(n_chunks):
            b = cc % 2
            gathers[cc].wait()
            if cc >= 1:
                scatters[cc - 1].wait()
            if cc + 1 < n_chunks:
                gathers[cc + 1] = gather_start(cc + 1)

            buf = bufs[b]
            obuf = obufs[b]

            @plsc.parallel_loop(0, CHUNK // 2, unroll=4)
            def _scale(j2, _buf=buf, _obuf=obuf):
                for h in range(2):
                    for r in range(vregs_per_row):
                        _obuf[j2, pl.ds(h * D + r * _L, _L)] = (
                            _buf[2 * j2 + h, pl.ds(r * _L, _L)] * SCALE_
                        )

            scatters[cc] = scatter_start(cc)
        scatters[n_chunks - 1].wait()

    return k


@jax.jit
def kernel(x, table):
    B = x.shape[0] * x.shape[1]
    V, D = table.shape
    flat_idx = x.reshape(-1).astype(jnp.int32)
    out2 = _make_sc_embed(B, V, D, 400)(flat_idx, table)
    return out2.reshape(x.shape[0], x.shape[1], D)


# speedup vs baseline: 1.4120x; 1.0464x over previous
import functools

import jax
import jax.numpy as jnp
from jax import lax
from jax.experimental import pallas as pl
from jax.experimental.pallas import tpu as pltpu
from jax.experimental.pallas import tpu_sc as plsc

SCALE_ = 8.0

_info = plsc.get_sparse_core_info()
_NC, _NS, _L = _info.num_cores, _info.num_subcores, _info.num_lanes
_NW = _NC * _NS


def _make_sc_embed(B, V, D, CHUNK):
    assert B % _NW == 0
    b_per_w = B // _NW
    assert b_per_w % CHUNK == 0
    n_chunks = b_per_w // CHUNK
    assert CHUNK % 16 == 0
    vregs_per_row = D // _L

    mesh = plsc.VectorSubcoreMesh(core_axis_name="c", subcore_axis_name="s")

    @functools.partial(
        pl.kernel,
        mesh=mesh,
        out_type=jax.ShapeDtypeStruct((B // 2, 2 * D), jnp.float32),
        scratch_types=[
            pltpu.VMEM((b_per_w,), jnp.int32),
            pltpu.VMEM((CHUNK, 2 * D), jnp.float32),
            pltpu.VMEM((CHUNK, 2 * D), jnp.float32),
            pltpu.VMEM((CHUNK // 2, 2 * D), jnp.float32),
            pltpu.VMEM((CHUNK // 2, 2 * D), jnp.float32),
            pltpu.SemaphoreType.DMA,
            pltpu.SemaphoreType.DMA,
            pltpu.SemaphoreType.DMA,
            pltpu.SemaphoreType.DMA,
        ],
        compiler_params=pltpu.CompilerParams(use_tc_tiling_on_sc=False),
    )
    def k(idx_hbm, table_hbm, out2_hbm, idx_all, rows0, rows1, ob0, ob1,
          g0, g1, s0, s1):
        wid = lax.axis_index("s") * _NC + lax.axis_index("c")
        base = wid * b_per_w
        pltpu.sync_copy(idx_hbm.at[pl.ds(base, b_per_w)], idx_all)

        bufs = (rows0, rows1)
        obufs = (ob0, ob1)
        gsem = (g0, g1)
        ssem = (s0, s1)

        def gather_start(cc):
            b = cc % 2
            return pltpu.async_copy(
                table_hbm.at[idx_all.at[pl.ds(cc * CHUNK, CHUNK)]],
                bufs[b],
                gsem[b],
            )

        def scatter_start(cc):
            b = cc % 2
            return pltpu.async_copy(
                obufs[b],
                out2_hbm.at[pl.ds((base + cc * CHUNK) // 2, CHUNK // 2)],
                ssem[b],
            )

        gathers = {0: gather_start(0)}
        scatters = {}
        for cc in range(n_chunks):
            b = cc % 2
            gathers[cc].wait()
            if cc >= 1:
                scatters[cc - 1].wait()
            if cc + 1 < n_chunks:
                gathers[cc + 1] = gather_start(cc + 1)

            buf = bufs[b]
            obuf = obufs[b]

            @plsc.parallel_loop(0, CHUNK // 2, unroll=4)
            def _scale(j2, _buf=buf, _obuf=obuf):
                for h in range(2):
                    for r in range(vregs_per_row):
                        _obuf[j2, pl.ds(h * D + r * _L, _L)] = (
                            _buf[2 * j2 + h, pl.ds(r * _L, _L)] * SCALE_
                        )
            del _scale

            scatters[cc] = scatter_start(cc)
        scatters[n_chunks - 1].wait()

    return k


@jax.jit
def kernel(x, table):
    B = x.shape[0] * x.shape[1]
    V, D = table.shape
    flat_idx = x.reshape(-1).astype(jnp.int32)
    table128 = jnp.pad(table, ((0, 0), (0, D)))
    out2 = _make_sc_embed(B, V, D, 256)(flat_idx, table128)
    return out2.reshape(x.shape[0], x.shape[1], D)
